# Initial kernel scaffold; baseline (speedup 1.0000x reference)
import jax
import jax.numpy as jnp
from jax.experimental import pallas as pl


def kernel(x, W_qkv, W_proj, b_proj):
    raise NotImplementedError("write your pallas kernel here")


# speedup vs baseline: 43.4516x; 43.4516x over previous
import jax
import jax.numpy as jnp
from jax.experimental import pallas as pl
from jax.experimental.pallas import tpu as pltpu


def _wadj_kernel(x_ref, out_ref, xt_ref, e_ref, c_ref, r_ref, wa_ref):
    n = out_ref.shape[1]
    c = x_ref.shape[2]
    scale = float(c) ** -0.5
    xb = x_ref[0]
    xt_ref[...] = xb.T

    sq = jnp.sum(xb * xb, axis=1)
    g = jax.lax.dot_general(xb, xb, (((1,), (1,)), ((), ())),
                            preferred_element_type=jnp.float32)
    d2 = sq[:, None] + sq[None, :] - 2.0 * g
    e_ref[...] = jnp.sqrt(jnp.maximum(d2, 1e-24))

    xc = xb - jnp.mean(xb, axis=1, keepdims=True)
    cov = jax.lax.dot_general(xc, xc, (((1,), (1,)), ((), ())),
                              preferred_element_type=jnp.float32) * (1.0 / (c - 1))
    ii = jax.lax.broadcasted_iota(jnp.int32, (n, n), 0)
    jj = jax.lax.broadcasted_iota(jnp.int32, (n, n), 1)
    diag = jnp.sum(jnp.where(ii == jj, cov, 0.0), axis=1)
    dstd = jnp.sqrt(jnp.clip(diag, 0.0, None))
    r_ref[...] = jnp.clip(cov / (dstd[:, None] * dstd[None, :]), -1.0, 1.0)

    m = None
    for ci in range(c // 8):
        rows = xt_ref[ci * 8:(ci + 1) * 8, :]
        cols = xb[:, ci * 8:(ci + 1) * 8]
        for f in range(8):
            t = jnp.abs(cols[:, f:f + 1] - rows[f:f + 1, :])
            m = t if m is None else jnp.maximum(m, t)
    c_ref[...] = m

    logits = jax.lax.dot_general(e_ref[...], c_ref[...], (((1,), (1,)), ((), ())),
                                 preferred_element_type=jnp.float32) * scale
    logits = logits - jnp.max(logits, axis=1, keepdims=True)
    p = jnp.exp(logits)
    p = p / jnp.sum(p, axis=1, keepdims=True)
    wa_ref[...] = jax.lax.dot_general(p, r_ref[...], (((1,), (0,)), ((), ())),
                                      preferred_element_type=jnp.float32)

    keep = (n * n) // 6

    def bisect_val(_, lohi):
        lo, hi = lohi
        mid = 0.5 * (lo + hi)
        cnt = jnp.sum((wa_ref[...] >= mid).astype(jnp.float32))
        ok = cnt >= keep
        return jnp.where(ok, mid, lo), jnp.where(ok, hi, mid)

    lo, _ = jax.lax.fori_loop(
        0, 50, bisect_val, (jnp.float32(-1.5), jnp.float32(1.5)))

    wa = wa_ref[...]
    tstar = jnp.min(jnp.where(wa >= lo, wa, jnp.float32(2.0)))
    cnt_gt = jnp.sum((wa > tstar).astype(jnp.float32))
    need = jnp.float32(keep) - cnt_gt

    fidx = (ii * n + jj).astype(jnp.float32)
    is_tie = (wa == tstar).astype(jnp.float32)

    def bisect_idx(_, lohi):
        lo_i, hi_i = lohi
        mid = jnp.floor(0.5 * (lo_i + hi_i))
        cnt = jnp.sum(is_tie * (fidx >= mid).astype(jnp.float32))
        ok = cnt >= need
        return jnp.where(ok, mid, lo_i), jnp.where(ok, hi_i, mid)

    cut, _ = jax.lax.fori_loop(
        0, 22, bisect_idx, (jnp.float32(0.0), jnp.float32(n * n)))

    marked = (wa > tstar) | ((wa == tstar) & (fidx >= cut))
    out_ref[0] = marked.astype(jnp.float32)


def kernel(x, W_qkv, W_proj, b_proj):
    B, N, C = x.shape
    return pl.pallas_call(
        _wadj_kernel,
        grid=(B,),
        in_specs=[pl.BlockSpec((1, N, C), lambda b: (b, 0, 0))],
        out_specs=pl.BlockSpec((1, N, N), lambda b: (b, 0, 0)),
        out_shape=jax.ShapeDtypeStruct((B, N, N), jnp.float32),
        scratch_shapes=[
            pltpu.VMEM((C, N), jnp.float32),
            pltpu.VMEM((N, N), jnp.float32),
            pltpu.VMEM((N, N), jnp.float32),
            pltpu.VMEM((N, N), jnp.float32),
            pltpu.VMEM((N, N), jnp.float32),
        ],
    )(x)


# speedup vs baseline: 50.1752x; 1.1547x over previous
import jax
import jax.numpy as jnp
from jax.experimental import pallas as pl
from jax.experimental.pallas import tpu as pltpu


def _wadj_kernel(x_ref, out_ref, xt_ref, e_ref, c_ref, r_ref, wa_ref):
    n = out_ref.shape[1]
    c = x_ref.shape[2]
    scale = float(c) ** -0.5
    xb = x_ref[0]
    xt_ref[...] = xb.T

    sq = jnp.sum(xb * xb, axis=1)
    g = jax.lax.dot_general(xb, xb, (((1,), (1,)), ((), ())),
                            preferred_element_type=jnp.float32)
    d2 = sq[:, None] + sq[None, :] - 2.0 * g
    e_ref[...] = jnp.sqrt(jnp.maximum(d2, 1e-24))

    xc = xb - jnp.mean(xb, axis=1, keepdims=True)
    cov = jax.lax.dot_general(xc, xc, (((1,), (1,)), ((), ())),
                              preferred_element_type=jnp.float32) * (1.0 / (c - 1))
    ii = jax.lax.broadcasted_iota(jnp.int32, (n, n), 0)
    jj = jax.lax.broadcasted_iota(jnp.int32, (n, n), 1)
    diag = jnp.sum(jnp.where(ii == jj, cov, 0.0), axis=1)
    dstd = jnp.sqrt(jnp.clip(diag, 0.0, None))
    r_ref[...] = jnp.clip(cov / (dstd[:, None] * dstd[None, :]), -1.0, 1.0)

    m = None
    for ci in range(c // 8):
        rows = xt_ref[ci * 8:(ci + 1) * 8, :]
        cols = xb[:, ci * 8:(ci + 1) * 8]
        for f in range(8):
            t = jnp.abs(cols[:, f:f + 1] - rows[f:f + 1, :])
            m = t if m is None else jnp.maximum(m, t)
    c_ref[...] = m

    logits = jax.lax.dot_general(e_ref[...], c_ref[...], (((1,), (1,)), ((), ())),
                                 preferred_element_type=jnp.float32) * scale
    logits = logits - jnp.max(logits, axis=1, keepdims=True)
    p = jnp.exp(logits)
    p = p / jnp.sum(p, axis=1, keepdims=True)
    wa_ref[...] = jax.lax.dot_general(p, r_ref[...], (((1,), (0,)), ((), ())),
                                      preferred_element_type=jnp.float32)

    keep = (n * n) // 6

    kf = jnp.float32(keep)

    def bisect_val(_, lohi):
        lo, hi = lohi
        q = 0.25 * (hi - lo)
        t1, t2, t3 = lo + q, lo + 2.0 * q, lo + 3.0 * q
        w = wa_ref[...]
        c1 = jnp.sum(jnp.where(w >= t1, 1.0, 0.0))
        c2 = jnp.sum(jnp.where(w >= t2, 1.0, 0.0))
        c3 = jnp.sum(jnp.where(w >= t3, 1.0, 0.0))
        lo2 = jnp.where(c3 >= kf, t3,
                        jnp.where(c2 >= kf, t2, jnp.where(c1 >= kf, t1, lo)))
        hi2 = jnp.where(c3 >= kf, hi,
                        jnp.where(c2 >= kf, t3, jnp.where(c1 >= kf, t2, t1)))
        return lo2, hi2

    lo, _ = jax.lax.fori_loop(
        0, 15, bisect_val, (jnp.float32(-1.5), jnp.float32(1.5)))

    wa = wa_ref[...]
    tstar = jnp.min(jnp.where(wa >= lo, wa, jnp.float32(2.0)))
    cnt_gt = jnp.sum((wa > tstar).astype(jnp.float32))
    need = jnp.float32(keep) - cnt_gt

    fidx = (ii * n + jj).astype(jnp.float32)
    is_tie = (wa == tstar).astype(jnp.float32)

    def bisect_idx(_, lohi):
        lo_i, hi_i = lohi
        q = 0.25 * (hi_i - lo_i)
        t1 = jnp.floor(lo_i + q)
        t2 = jnp.floor(lo_i + 2.0 * q)
        t3 = jnp.floor(lo_i + 3.0 * q)
        c1 = jnp.sum(is_tie * jnp.where(fidx >= t1, 1.0, 0.0))
        c2 = jnp.sum(is_tie * jnp.where(fidx >= t2, 1.0, 0.0))
        c3 = jnp.sum(is_tie * jnp.where(fidx >= t3, 1.0, 0.0))
        lo2 = jnp.where(c3 >= need, t3,
                        jnp.where(c2 >= need, t2, jnp.where(c1 >= need, t1, lo_i)))
        hi2 = jnp.where(c3 >= need, hi_i,
                        jnp.where(c2 >= need, t3, jnp.where(c1 >= need, t2, t1)))
        return lo2, hi2

    cut, _ = jax.lax.fori_loop(
        0, 14, bisect_idx, (jnp.float32(0.0), jnp.float32(n * n)))

    marked = (wa > tstar) | ((wa == tstar) & (fidx >= cut))
    out_ref[0] = marked.astype(jnp.float32)


def kernel(x, W_qkv, W_proj, b_proj):
    B, N, C = x.shape
    return pl.pallas_call(
        _wadj_kernel,
        grid=(B,),
        in_specs=[pl.BlockSpec((1, N, C), lambda b: (b, 0, 0))],
        out_specs=pl.BlockSpec((1, N, N), lambda b: (b, 0, 0)),
        out_shape=jax.ShapeDtypeStruct((B, N, N), jnp.float32),
        scratch_shapes=[
            pltpu.VMEM((C, N), jnp.float32),
            pltpu.VMEM((N, N), jnp.float32),
            pltpu.VMEM((N, N), jnp.float32),
            pltpu.VMEM((N, N), jnp.float32),
            pltpu.VMEM((N, N), jnp.float32),
        ],
    )(x)


# speedup vs baseline: 50.1963x; 1.0004x over previous
import jax
import jax.numpy as jnp
from jax.experimental import pallas as pl
from jax.experimental.pallas import tpu as pltpu


def _wadj_kernel(x_ref, out_ref, xt_ref, e_ref, c_ref, r_ref, wa_ref):
    n = out_ref.shape[1]
    c = x_ref.shape[2]
    scale = float(c) ** -0.5
    xb = x_ref[0]
    xt_ref[...] = xb.T

    sq = jnp.sum(xb * xb, axis=1)
    g = jax.lax.dot_general(xb, xb, (((1,), (1,)), ((), ())),
                            preferred_element_type=jnp.float32)
    d2 = sq[:, None] + sq[None, :] - 2.0 * g
    e_ref[...] = jnp.sqrt(jnp.maximum(d2, 1e-24))

    xc = xb - jnp.mean(xb, axis=1, keepdims=True)
    cov = jax.lax.dot_general(xc, xc, (((1,), (1,)), ((), ())),
                              preferred_element_type=jnp.float32) * (1.0 / (c - 1))
    ii = jax.lax.broadcasted_iota(jnp.int32, (n, n), 0)
    jj = jax.lax.broadcasted_iota(jnp.int32, (n, n), 1)
    diag = jnp.sum(jnp.where(ii == jj, cov, 0.0), axis=1)
    dstd = jnp.sqrt(jnp.clip(diag, 0.0, None))
    r_ref[...] = jnp.clip(cov / (dstd[:, None] * dstd[None, :]), -1.0, 1.0)

    m = None
    for ci in range(c // 8):
        rows = xt_ref[ci * 8:(ci + 1) * 8, :]
        cols = xb[:, ci * 8:(ci + 1) * 8]
        for f in range(8):
            t = jnp.abs(cols[:, f:f + 1] - rows[f:f + 1, :])
            m = t if m is None else jnp.maximum(m, t)
    c_ref[...] = m

    logits = jax.lax.dot_general(e_ref[...], c_ref[...], (((1,), (1,)), ((), ())),
                                 preferred_element_type=jnp.float32) * scale
    logits = logits - jnp.max(logits, axis=1, keepdims=True)
    p = jnp.exp(logits)
    p = p / jnp.sum(p, axis=1, keepdims=True)
    wa_ref[...] = jax.lax.dot_general(p, r_ref[...], (((1,), (0,)), ((), ())),
                                      preferred_element_type=jnp.float32)

    keep = (n * n) // 6

    kf = jnp.float32(keep)

    def bisect_val(_, lohi):
        lo, hi = lohi
        q = 0.25 * (hi - lo)
        t1, t2, t3 = lo + q, lo + 2.0 * q, lo + 3.0 * q
        w = wa_ref[...]
        c1 = jnp.sum(jnp.where(w >= t1, 1.0, 0.0))
        c2 = jnp.sum(jnp.where(w >= t2, 1.0, 0.0))
        c3 = jnp.sum(jnp.where(w >= t3, 1.0, 0.0))
        lo2 = jnp.where(c3 >= kf, t3,
                        jnp.where(c2 >= kf, t2, jnp.where(c1 >= kf, t1, lo)))
        hi2 = jnp.where(c3 >= kf, hi,
                        jnp.where(c2 >= kf, t3, jnp.where(c1 >= kf, t2, t1)))
        return lo2, hi2

    lo, _ = jax.lax.fori_loop(
        0, 15, bisect_val, (jnp.float32(-1.5), jnp.float32(1.5)))

    wa = wa_ref[...]
    tstar = jnp.min(jnp.where(wa >= lo, wa, jnp.float32(2.0)))
    cnt_gt = jnp.sum((wa > tstar).astype(jnp.float32))
    need = jnp.float32(keep) - cnt_gt

    fidx = (ii * n + jj).astype(jnp.float32)
    is_tie = (wa == tstar).astype(jnp.float32)

    def bisect_idx(_, lohi):
        lo_i, hi_i = lohi
        q = 0.25 * (hi_i - lo_i)
        t1 = jnp.floor(lo_i + q)
        t2 = jnp.floor(lo_i + 2.0 * q)
        t3 = jnp.floor(lo_i + 3.0 * q)
        c1 = jnp.sum(is_tie * jnp.where(fidx >= t1, 1.0, 0.0))
        c2 = jnp.sum(is_tie * jnp.where(fidx >= t2, 1.0, 0.0))
        c3 = jnp.sum(is_tie * jnp.where(fidx >= t3, 1.0, 0.0))
        lo2 = jnp.where(c3 >= need, t3,
                        jnp.where(c2 >= need, t2, jnp.where(c1 >= need, t1, lo_i)))
        hi2 = jnp.where(c3 >= need, hi_i,
                        jnp.where(c2 >= need, t3, jnp.where(c1 >= need, t2, t1)))
        return lo2, hi2

    cut, _ = jax.lax.fori_loop(
        0, 14, bisect_idx, (jnp.float32(0.0), jnp.float32(n * n)))

    marked = (wa > tstar) | ((wa == tstar) & (fidx >= cut))
    out_ref[0] = marked.astype(jnp.float32)


def kernel(x, W_qkv, W_proj, b_proj):
    B, N, C = x.shape
    return pl.pallas_call(
        _wadj_kernel,
        grid=(B,),
        in_specs=[pl.BlockSpec((1, N, C), lambda b: (b, 0, 0))],
        out_specs=pl.BlockSpec((1, N, N), lambda b: (b, 0, 0)),
        out_shape=jax.ShapeDtypeStruct((B, N, N), jnp.float32),
        scratch_shapes=[
            pltpu.VMEM((C, N), jnp.float32),
            pltpu.VMEM((N, N), jnp.float32),
            pltpu.VMEM((N, N), jnp.float32),
            pltpu.VMEM((N, N), jnp.float32),
            pltpu.VMEM((N, N), jnp.float32),
        ],
        compiler_params=pltpu.CompilerParams(
            dimension_semantics=("parallel",)),
    )(x)
